# baseline (speedup 1.0000x reference)
import jax
import jax.numpy as jnp
from jax import lax
from jax.experimental import pallas as pl
from jax.experimental.pallas import tpu as pltpu

_M32 = 0xFFFFFFFF


def _py_threefry2x32(k0, k1, x0, x1):
    ks = (k0, k1, (k0 ^ k1 ^ 0x1BD11BDA) & _M32)
    rots = ((13, 15, 26, 6), (17, 29, 16, 24))
    v0 = (x0 + ks[0]) & _M32
    v1 = (x1 + ks[1]) & _M32
    for i in range(5):
        for r in rots[i % 2]:
            v0 = (v0 + v1) & _M32
            v1 = ((v1 << r) | (v1 >> (32 - r))) & _M32
            v1 ^= v0
        v0 = (v0 + ks[(i + 1) % 3]) & _M32
        v1 = (v1 + ks[(i + 2) % 3] + i + 1) & _M32
    return v0, v1


_K0, _K1 = _py_threefry2x32(0, 0, 0, 1)
_K2 = (_K0 ^ _K1 ^ 0x1BD11BDA) & _M32


def _tf_bits(flat):
    ks = (jnp.uint32(_K0), jnp.uint32(_K1), jnp.uint32(_K2))
    rots = ((13, 15, 26, 6), (17, 29, 16, 24))
    x0 = jnp.full_like(flat, ks[0])
    x1 = flat + ks[1]
    for i in range(5):
        for r in rots[i % 2]:
            x0 = x0 + x1
            x1 = (x1 << r) | (x1 >> (32 - r))
            x1 = x1 ^ x0
        x0 = x0 + ks[(i + 1) % 3]
        x1 = x1 + ks[(i + 2) % 3] + jnp.uint32(i + 1)
    return x0 ^ x1


import numpy as np


def _make_body(rows, cols, blk, nb):
    minv = np.float32(1e-20)
    span = np.float32(np.float32(1.0) - minv)

    def body(x_ref, o_ref, bv, bi):
        j = pl.program_id(0)

        @pl.when(j == 0)
        def _init():
            bv[...] = jnp.full((rows, 1), -jnp.inf, jnp.float32)
            bi[...] = jnp.zeros((rows, 1), jnp.int32)

        col = lax.broadcasted_iota(jnp.int32, (rows, blk), 1) + j * blk
        row = lax.broadcasted_iota(jnp.int32, (rows, blk), 0)
        flat = (row * cols + col).astype(jnp.uint32)
        bits = _tf_bits(flat)
        fb = lax.bitcast_convert_type(
            (bits >> jnp.uint32(9)) | jnp.uint32(0x3F800000), jnp.float32)
        u = jnp.maximum(minv, (fb - np.float32(1.0)) * span + minv)
        g = -jnp.log(-jnp.log(u))
        phi = jnp.where(col < cols, x_ref[...] + g, -jnp.inf)
        m = jnp.max(phi, axis=1, keepdims=True)
        idx = jnp.min(jnp.where(phi == m, col, jnp.int32(0x7FFFFFFF)),
                      axis=1, keepdims=True)
        better = m > bv[...]
        bv[...] = jnp.where(better, m, bv[...])
        bi[...] = jnp.where(better, idx, bi[...])

        @pl.when(j == nb - 1)
        def _fin():
            o_ref[...] = bi[...]

    return body


def kernel(logits):
    rows, cols = logits.shape
    blk = min(cols, 4096)
    nb = (cols + blk - 1) // blk
    out = pl.pallas_call(
        _make_body(rows, cols, blk, nb),
        grid=(nb,),
        in_specs=[pl.BlockSpec((rows, blk), lambda j: (0, j))],
        out_specs=pl.BlockSpec((rows, 1), lambda j: (0, 0)),
        out_shape=jax.ShapeDtypeStruct((rows, 1), jnp.int32),
        scratch_shapes=[pltpu.VMEM((rows, 1), jnp.float32),
                        pltpu.VMEM((rows, 1), jnp.int32)],
        compiler_params=pltpu.CompilerParams(
            dimension_semantics=("arbitrary",)),
    )(logits)
    return out.reshape(rows).astype(jnp.int64)
